# baseline (speedup 1.0000x reference)
import functools
import numpy as np
import jax
import jax.numpy as jnp
from jax import lax
from jax.experimental import pallas as pl
from jax.experimental.pallas import tpu as pltpu
from jax.experimental.pallas import tpu_sc as plsc

NSAMPLE = 32
TM = 128
TN = 2048
EPS = 1e-5


NSEG = 32
SEGW = 256
NPOP = 4
JPASS = 8
SLOTS = 32
T1PASS = 2
INF = np.float32(np.inf)


def _topk_kernel(xyz4_ref, cen_ref, idx_ref, v3_ref, bufv_ref, bufi_ref,
                 *, P, k):
    b = pl.program_id(0)
    f32 = jnp.float32
    bf = jnp.bfloat16

    px = xyz4_ref[0, 0]
    py = xyz4_ref[0, 1]
    pz = xyz4_ref[0, 2]
    c2 = (px * px + py * py + pz * pz).reshape(NSEG, 1, SEGW)
    cx = cen_ref[0, :, 0].reshape(1, TM, 1)
    cy = cen_ref[0, :, 1].reshape(1, TM, 1)
    cz = cen_ref[0, :, 2].reshape(1, TM, 1)
    a2 = cx * cx + cy * cy + cz * cz
    dot = (cx.astype(bf).astype(f32) * px.astype(bf).astype(f32).reshape(NSEG, 1, SEGW)
           + cy.astype(bf).astype(f32) * py.astype(bf).astype(f32).reshape(NSEG, 1, SEGW)
           + cz.astype(bf).astype(f32) * pz.astype(bf).astype(f32).reshape(NSEG, 1, SEGW))
    v3_ref[...] = jnp.sqrt(jnp.maximum(a2 + c2 - 2.0 * dot, 0.0))

    bufv_ref[...] = jnp.full((SLOTS, NSEG, TM), INF, f32)

    lane_iota = lax.broadcasted_iota(jnp.int32, (NSEG, TM, SEGW), 2)
    seg_iota = lax.broadcasted_iota(jnp.int32, (NSEG, TM), 0)

    def pop_once(v, smin):
        eq = v == smin[:, :, None]
        sidx = jnp.min(jnp.where(eq, lane_iota, SEGW), axis=2)
        vnew = jnp.where(eq & (lane_iota == sidx[:, :, None]), INF, v)
        return vnew, seg_iota * SEGW + sidx

    def cond(c):
        j, done = c
        return jnp.logical_and(j < JPASS, jnp.logical_not(done))

    def body(c):
        j, _ = c
        v = v3_ref[...]
        smin1 = jnp.min(v, axis=2)
        remmin = jnp.min(smin1, axis=0)
        cnt = jnp.sum((bufv_ref[...] < remmin[None, None, :]).astype(jnp.int32),
                      axis=(0, 1))
        done = jnp.all(cnt >= k)

        @pl.when(jnp.logical_not(done))
        def _append():
            vc = v
            smin = smin1
            for t in range(NPOP):
                vc, gidx = pop_once(vc, smin)
                bufv_ref[pl.ds(NPOP * j + t, 1)] = smin[None]
                bufi_ref[pl.ds(NPOP * j + t, 1)] = gidx[None]
                if t + 1 < NPOP:
                    smin = jnp.min(vc, axis=2)
            v3_ref[...] = vc

        return j + 1, done

    j_final, _ = lax.while_loop(cond, body, (jnp.int32(0), jnp.bool_(False)))

    def merge(bv, bi):
        cols = []
        for _ in range(k):
            m = jnp.min(bv, axis=(0, 1))
            sel = bv == m[None, None, :]
            am = jnp.min(jnp.where(sel, bi, P), axis=(0, 1))
            cols.append((am + b * P).reshape(1, TM))
            bv = jnp.where(sel & (bi == am[None, None, :]), INF, bv)
        idx_ref[0, :, :] = jnp.concatenate(cols, axis=0)

    @pl.when(j_final <= T1PASS)
    def _fast():
        merge(bufv_ref[0:NPOP * T1PASS], bufi_ref[0:NPOP * T1PASS])

    @pl.when(j_final > T1PASS)
    def _slow():
        merge(bufv_ref[...], bufi_ref[...])


def _run_topk(xyz4, centers, B, M, P, k):
    grid = (B, M // TM)
    return pl.pallas_call(
        functools.partial(_topk_kernel, P=P, k=k),
        grid=grid,
        in_specs=[
            pl.BlockSpec((1, 3, NSEG, SEGW), lambda b, mt: (b, 0, 0, 0)),
            pl.BlockSpec((1, TM, 3), lambda b, mt: (b, mt, 0)),
        ],
        out_specs=pl.BlockSpec((1, k, TM), lambda b, mt: (b, 0, mt)),
        out_shape=jax.ShapeDtypeStruct((B, k, M), jnp.int32),
        scratch_shapes=[
            pltpu.VMEM((NSEG, TM, SEGW), jnp.float32),
            pltpu.VMEM((SLOTS, NSEG, TM), jnp.float32),
            pltpu.VMEM((SLOTS, NSEG, TM), jnp.int32),
        ],
    )(xyz4, centers)


def _run_gather(table, idx_flat, n_rows, D):
    info = plsc.get_sparse_core_info()
    NW = info.num_cores * info.num_subcores
    CH = 128
    per_w = n_rows // NW
    n_ch = per_w // CH
    mesh = plsc.VectorSubcoreMesh(core_axis_name="c", subcore_axis_name="s")

    @functools.partial(
        pl.kernel, mesh=mesh,
        out_type=jax.ShapeDtypeStruct((n_rows, D), jnp.float32),
        scratch_types=[
            pltpu.VMEM((CH,), jnp.int32),
            pltpu.VMEM((CH, D), jnp.float32),
            pltpu.SemaphoreType.DMA,
        ],
    )
    def gather_k(table_hbm, idx_hbm, out_hbm, idx_v, rows_v, sem):
        wid = lax.axis_index("s") * info.num_cores + lax.axis_index("c")
        base = wid * per_w

        def body(i, _):
            off = base + i * CH
            pltpu.sync_copy(idx_hbm.at[pl.ds(off, CH)], idx_v)
            pltpu.async_copy(table_hbm.at[idx_v], rows_v, sem).wait()
            pltpu.sync_copy(rows_v, out_hbm.at[pl.ds(off, CH)])
            return 0

        lax.fori_loop(0, n_ch, body, 0)

    return gather_k(table, idx_flat)


def _stats1_kernel(x_ref, crep_ref, w1p_ref, w1xT_ref, b1_ref, s_ref):
    y1 = (jnp.dot(x_ref[...], w1p_ref[...], preferred_element_type=jnp.float32)
          - jnp.dot(crep_ref[...], w1xT_ref[...], preferred_element_type=jnp.float32)
          + b1_ref[...])
    s = jnp.sum(y1, axis=0, keepdims=True)
    ss = jnp.sum(y1 * y1, axis=0, keepdims=True)
    tile = jnp.concatenate([s, ss], axis=0)

    @pl.when(pl.program_id(0) == 0)
    def _init():
        s_ref[...] = jnp.zeros_like(s_ref)

    s_ref[...] += tile


def _run_stats1(x, crep, W1pad, W1xT, b1, N, D, C1):
    grid = (N // TN,)
    return pl.pallas_call(
        _stats1_kernel,
        grid=grid,
        in_specs=[
            pl.BlockSpec((TN, D), lambda i: (i, 0)),
            pl.BlockSpec((TN, 3), lambda i: (i, 0)),
            pl.BlockSpec((D, C1), lambda i: (0, 0)),
            pl.BlockSpec((3, C1), lambda i: (0, 0)),
            pl.BlockSpec((1, C1), lambda i: (0, 0)),
        ],
        out_specs=pl.BlockSpec((2, C1), lambda i: (0, 0)),
        out_shape=jax.ShapeDtypeStruct((2, C1), jnp.float32),
    )(x, crep, W1pad, W1xT, b1)


def _layer2_kernel(x_ref, crep_ref, w1p_ref, w1xT_ref, b1_ref, g1_ref, be1_ref,
                   s1_ref, w2T_ref, b2_ref, s2_ref, ymax_ref, ymin_ref,
                   *, N, k):
    s1 = s1_ref[0:1, :]
    ss1 = s1_ref[1:2, :]
    mean1 = s1 / N
    var1 = ss1 / N - mean1 * mean1
    a1 = g1_ref[...] * lax.rsqrt(var1 + EPS)
    d1 = be1_ref[...] - mean1 * a1

    y1 = (jnp.dot(x_ref[...], w1p_ref[...], preferred_element_type=jnp.float32)
          - jnp.dot(crep_ref[...], w1xT_ref[...], preferred_element_type=jnp.float32)
          + b1_ref[...])
    h = jnp.maximum(y1 * a1 + d1, 0.0)
    y2 = jnp.dot(h, w2T_ref[...], preferred_element_type=jnp.float32) + b2_ref[...]

    s = jnp.sum(y2, axis=0, keepdims=True)
    ss = jnp.sum(y2 * y2, axis=0, keepdims=True)
    tile = jnp.concatenate([s, ss], axis=0)

    @pl.when(pl.program_id(0) == 0)
    def _init():
        s2_ref[...] = jnp.zeros_like(s2_ref)

    s2_ref[...] += tile

    C2 = y2.shape[-1]
    y2g = y2.reshape(TN // k, k, C2)
    ymax_ref[...] = jnp.max(y2g, axis=1)
    ymin_ref[...] = jnp.min(y2g, axis=1)


def _run_layer2(x, crep, W1pad, W1xT, b1, g1, be1, s1, W2T, b2, N, D, C1, C2, k):
    grid = (N // TN,)
    return pl.pallas_call(
        functools.partial(_layer2_kernel, N=N, k=k),
        grid=grid,
        in_specs=[
            pl.BlockSpec((TN, D), lambda i: (i, 0)),
            pl.BlockSpec((TN, 3), lambda i: (i, 0)),
            pl.BlockSpec((D, C1), lambda i: (0, 0)),
            pl.BlockSpec((3, C1), lambda i: (0, 0)),
            pl.BlockSpec((1, C1), lambda i: (0, 0)),
            pl.BlockSpec((1, C1), lambda i: (0, 0)),
            pl.BlockSpec((1, C1), lambda i: (0, 0)),
            pl.BlockSpec((2, C1), lambda i: (0, 0)),
            pl.BlockSpec((C1, C2), lambda i: (0, 0)),
            pl.BlockSpec((1, C2), lambda i: (0, 0)),
        ],
        out_specs=[
            pl.BlockSpec((2, C2), lambda i: (0, 0)),
            pl.BlockSpec((TN // k, C2), lambda i: (i, 0)),
            pl.BlockSpec((TN // k, C2), lambda i: (i, 0)),
        ],
        out_shape=[
            jax.ShapeDtypeStruct((2, C2), jnp.float32),
            jax.ShapeDtypeStruct((N // k, C2), jnp.float32),
            jax.ShapeDtypeStruct((N // k, C2), jnp.float32),
        ],
    )(x, crep, W1pad, W1xT, b1, g1, be1, s1, W2T, b2)


def _final_kernel(ymax_ref, ymin_ref, s2_ref, g2_ref, be2_ref, out_ref, *, N):
    s2 = s2_ref[0:1, :]
    ss2 = s2_ref[1:2, :]
    mean2 = s2 / N
    var2 = ss2 / N - mean2 * mean2
    a2 = g2_ref[...] * lax.rsqrt(var2 + EPS)
    d2 = be2_ref[...] - mean2 * a2
    sel = jnp.where(a2 >= 0.0, ymax_ref[...], ymin_ref[...])
    out_ref[...] = jnp.maximum(sel * a2 + d2, 0.0)


def _run_final(ymax, ymin, s2, g2, be2, N, C2, TF=1024):
    M_all = ymax.shape[0]
    grid = (M_all // TF,)
    return pl.pallas_call(
        functools.partial(_final_kernel, N=N),
        grid=grid,
        in_specs=[
            pl.BlockSpec((TF, C2), lambda i: (i, 0)),
            pl.BlockSpec((TF, C2), lambda i: (i, 0)),
            pl.BlockSpec((2, C2), lambda i: (0, 0)),
            pl.BlockSpec((1, C2), lambda i: (0, 0)),
            pl.BlockSpec((1, C2), lambda i: (0, 0)),
        ],
        out_specs=pl.BlockSpec((TF, C2), lambda i: (i, 0)),
        out_shape=jax.ShapeDtypeStruct((M_all, C2), jnp.float32),
    )(ymax, ymin, s2, g2, be2)


@jax.jit
def kernel(xyz, feats, W1, bi1, g1, be1, W2, bi2, g2, be2):
    B, P, _ = xyz.shape
    C = feats.shape[1]
    M = max(1, P // 4)
    k = min(NSAMPLE, P)
    N = B * M * k
    D = 128
    C1 = W1.shape[0]
    C2 = W2.shape[0]

    idx_center = jnp.linspace(0.0, float(P - 1), M).astype(jnp.int32)
    centers = xyz[:, idx_center, :]

    xyz4 = jnp.transpose(xyz, (0, 2, 1)).reshape(B, 3, NSEG, SEGW)
    idx_g = _run_topk(xyz4, centers, B, M, P, k)
    idx_g = jnp.transpose(idx_g, (0, 2, 1))

    feats_perm = jnp.transpose(feats, (0, 2, 1))
    table = jnp.concatenate(
        [feats_perm, xyz, jnp.zeros((B, P, D - C - 3), jnp.float32)], axis=2
    ).reshape(B * P, D)
    idx_flat = idx_g.reshape(N)
    x = _run_gather(table, idx_flat, N, D)

    W1pad = jnp.concatenate(
        [W1[:, 3:3 + C].T, W1[:, 0:3].T, jnp.zeros((D - C - 3, C1), jnp.float32)],
        axis=0)
    W1xT = W1[:, 0:3].T
    W2T = W2.T
    b1 = bi1.reshape(1, C1)
    g1r = g1.reshape(1, C1)
    be1r = be1.reshape(1, C1)
    b2 = bi2.reshape(1, C2)
    g2r = g2.reshape(1, C2)
    be2r = be2.reshape(1, C2)
    crep = jnp.repeat(centers.reshape(B * M, 3), k, axis=0)

    s1 = _run_stats1(x, crep, W1pad, W1xT, b1, N, D, C1)
    s2, ymax, ymin = _run_layer2(x, crep, W1pad, W1xT, b1, g1r, be1r, s1,
                                 W2T, b2, N, D, C1, C2, k)
    pooled = _run_final(ymax, ymin, s2, g2r, be2r, N, C2)

    out = jnp.transpose(pooled.reshape(B, M, C2), (0, 2, 1))
    return centers, out


# speedup vs baseline: 1.3128x; 1.3128x over previous
import functools
import numpy as np
import jax
import jax.numpy as jnp
from jax import lax
from jax.experimental import pallas as pl
from jax.experimental.pallas import tpu as pltpu
from jax.experimental.pallas import tpu_sc as plsc

NSAMPLE = 32
TM = 128
TN = 2048
EPS = 1e-5


NSEG = 32
SEGW = 256
NPOP = 2
JPASS = 16
SLOTS = 32
T1PASS = 4
INF = np.float32(np.inf)


def _topk_kernel(xyz4_ref, cen_ref, idx_ref, v3_ref, bufv_ref, bufi_ref,
                 *, P, k):
    b = pl.program_id(0)
    f32 = jnp.float32
    bf = jnp.bfloat16

    px = xyz4_ref[0, 0]
    py = xyz4_ref[0, 1]
    pz = xyz4_ref[0, 2]
    c2 = (px * px + py * py + pz * pz).reshape(NSEG, 1, SEGW)
    cx = cen_ref[0, :, 0].reshape(1, TM, 1)
    cy = cen_ref[0, :, 1].reshape(1, TM, 1)
    cz = cen_ref[0, :, 2].reshape(1, TM, 1)
    a2 = cx * cx + cy * cy + cz * cz
    dot = (cx.astype(bf).astype(f32) * px.astype(bf).astype(f32).reshape(NSEG, 1, SEGW)
           + cy.astype(bf).astype(f32) * py.astype(bf).astype(f32).reshape(NSEG, 1, SEGW)
           + cz.astype(bf).astype(f32) * pz.astype(bf).astype(f32).reshape(NSEG, 1, SEGW))
    v3_ref[...] = jnp.sqrt(jnp.maximum(a2 + c2 - 2.0 * dot, 0.0))

    bufv_ref[...] = jnp.full((SLOTS, NSEG, TM), INF, f32)

    lane_iota = lax.broadcasted_iota(jnp.int32, (NSEG, TM, SEGW), 2)
    seg_iota = lax.broadcasted_iota(jnp.int32, (NSEG, TM), 0)

    def pop_once(v, smin):
        eq = v == smin[:, :, None]
        sidx = jnp.min(jnp.where(eq, lane_iota, SEGW), axis=2)
        vnew = jnp.where(lane_iota == sidx[:, :, None], INF, v)
        return vnew, seg_iota * SEGW + sidx

    def cond(c):
        j, done = c
        return jnp.logical_and(j < JPASS, jnp.logical_not(done))

    def body(c):
        j, _ = c
        v = v3_ref[...]
        smin1 = jnp.min(v, axis=2)
        remmin = jnp.min(smin1, axis=0)
        cnt = jnp.sum((bufv_ref[...] < remmin[None, None, :]).astype(jnp.int32),
                      axis=(0, 1))
        done = jnp.all(cnt >= k)

        @pl.when(jnp.logical_not(done))
        def _append():
            vc = v
            smin = smin1
            for t in range(NPOP):
                vc, gidx = pop_once(vc, smin)
                bufv_ref[pl.ds(NPOP * j + t, 1)] = smin[None]
                bufi_ref[pl.ds(NPOP * j + t, 1)] = gidx[None]
                if t + 1 < NPOP:
                    smin = jnp.min(vc, axis=2)
            v3_ref[...] = vc

        return j + 1, done

    j_final, _ = lax.while_loop(cond, body, (jnp.int32(0), jnp.bool_(False)))

    def merge(bv, bi):
        cols = []
        for _ in range(k):
            m = jnp.min(bv, axis=(0, 1))
            am = jnp.min(jnp.where(bv == m[None, None, :], bi, P), axis=(0, 1))
            cols.append((am + b * P).reshape(1, TM))
            bv = jnp.where(bi == am[None, None, :], INF, bv)
        idx_ref[0, :, :] = jnp.concatenate(cols, axis=0)

    @pl.when(j_final <= T1PASS)
    def _fast():
        merge(bufv_ref[0:NPOP * T1PASS], bufi_ref[0:NPOP * T1PASS])

    @pl.when(j_final > T1PASS)
    def _slow():
        merge(bufv_ref[...], bufi_ref[...])


def _run_topk(xyz4, centers, B, M, P, k):
    grid = (B, M // TM)
    return pl.pallas_call(
        functools.partial(_topk_kernel, P=P, k=k),
        grid=grid,
        in_specs=[
            pl.BlockSpec((1, 3, NSEG, SEGW), lambda b, mt: (b, 0, 0, 0)),
            pl.BlockSpec((1, TM, 3), lambda b, mt: (b, mt, 0)),
        ],
        out_specs=pl.BlockSpec((1, k, TM), lambda b, mt: (b, 0, mt)),
        out_shape=jax.ShapeDtypeStruct((B, k, M), jnp.int32),
        scratch_shapes=[
            pltpu.VMEM((NSEG, TM, SEGW), jnp.float32),
            pltpu.VMEM((SLOTS, NSEG, TM), jnp.float32),
            pltpu.VMEM((SLOTS, NSEG, TM), jnp.int32),
        ],
    )(xyz4, centers)


def _run_gather(table, idx_flat, n_rows, D):
    info = plsc.get_sparse_core_info()
    NW = info.num_cores * info.num_subcores
    CH = 128
    per_w = n_rows // NW
    n_ch = per_w // CH
    mesh = plsc.VectorSubcoreMesh(core_axis_name="c", subcore_axis_name="s")

    @functools.partial(
        pl.kernel, mesh=mesh,
        out_type=jax.ShapeDtypeStruct((n_rows, D), jnp.float32),
        scratch_types=[
            pltpu.VMEM((CH,), jnp.int32),
            pltpu.VMEM((CH, D), jnp.float32),
            pltpu.SemaphoreType.DMA,
        ],
    )
    def gather_k(table_hbm, idx_hbm, out_hbm, idx_v, rows_v, sem):
        wid = lax.axis_index("s") * info.num_cores + lax.axis_index("c")
        base = wid * per_w

        def body(i, _):
            off = base + i * CH
            pltpu.sync_copy(idx_hbm.at[pl.ds(off, CH)], idx_v)
            pltpu.async_copy(table_hbm.at[idx_v], rows_v, sem).wait()
            pltpu.sync_copy(rows_v, out_hbm.at[pl.ds(off, CH)])
            return 0

        lax.fori_loop(0, n_ch, body, 0)

    return gather_k(table, idx_flat)


def _stats1_kernel(x_ref, crep_ref, w1p_ref, w1xT_ref, b1_ref, s_ref):
    y1 = (jnp.dot(x_ref[...], w1p_ref[...], preferred_element_type=jnp.float32)
          - jnp.dot(crep_ref[...], w1xT_ref[...], preferred_element_type=jnp.float32)
          + b1_ref[...])
    s = jnp.sum(y1, axis=0, keepdims=True)
    ss = jnp.sum(y1 * y1, axis=0, keepdims=True)
    tile = jnp.concatenate([s, ss], axis=0)

    @pl.when(pl.program_id(0) == 0)
    def _init():
        s_ref[...] = jnp.zeros_like(s_ref)

    s_ref[...] += tile


def _run_stats1(x, crep, W1pad, W1xT, b1, N, D, C1):
    grid = (N // TN,)
    return pl.pallas_call(
        _stats1_kernel,
        grid=grid,
        in_specs=[
            pl.BlockSpec((TN, D), lambda i: (i, 0)),
            pl.BlockSpec((TN, 3), lambda i: (i, 0)),
            pl.BlockSpec((D, C1), lambda i: (0, 0)),
            pl.BlockSpec((3, C1), lambda i: (0, 0)),
            pl.BlockSpec((1, C1), lambda i: (0, 0)),
        ],
        out_specs=pl.BlockSpec((2, C1), lambda i: (0, 0)),
        out_shape=jax.ShapeDtypeStruct((2, C1), jnp.float32),
    )(x, crep, W1pad, W1xT, b1)


def _layer2_kernel(x_ref, crep_ref, w1p_ref, w1xT_ref, b1_ref, g1_ref, be1_ref,
                   s1_ref, w2T_ref, b2_ref, s2_ref, ymax_ref, ymin_ref,
                   *, N, k):
    s1 = s1_ref[0:1, :]
    ss1 = s1_ref[1:2, :]
    mean1 = s1 / N
    var1 = ss1 / N - mean1 * mean1
    a1 = g1_ref[...] * lax.rsqrt(var1 + EPS)
    d1 = be1_ref[...] - mean1 * a1

    y1 = (jnp.dot(x_ref[...], w1p_ref[...], preferred_element_type=jnp.float32)
          - jnp.dot(crep_ref[...], w1xT_ref[...], preferred_element_type=jnp.float32)
          + b1_ref[...])
    h = jnp.maximum(y1 * a1 + d1, 0.0)
    y2 = jnp.dot(h, w2T_ref[...], preferred_element_type=jnp.float32) + b2_ref[...]

    s = jnp.sum(y2, axis=0, keepdims=True)
    ss = jnp.sum(y2 * y2, axis=0, keepdims=True)
    tile = jnp.concatenate([s, ss], axis=0)

    @pl.when(pl.program_id(0) == 0)
    def _init():
        s2_ref[...] = jnp.zeros_like(s2_ref)

    s2_ref[...] += tile

    C2 = y2.shape[-1]
    y2g = y2.reshape(TN // k, k, C2)
    ymax_ref[...] = jnp.max(y2g, axis=1)
    ymin_ref[...] = jnp.min(y2g, axis=1)


def _run_layer2(x, crep, W1pad, W1xT, b1, g1, be1, s1, W2T, b2, N, D, C1, C2, k):
    grid = (N // TN,)
    return pl.pallas_call(
        functools.partial(_layer2_kernel, N=N, k=k),
        grid=grid,
        in_specs=[
            pl.BlockSpec((TN, D), lambda i: (i, 0)),
            pl.BlockSpec((TN, 3), lambda i: (i, 0)),
            pl.BlockSpec((D, C1), lambda i: (0, 0)),
            pl.BlockSpec((3, C1), lambda i: (0, 0)),
            pl.BlockSpec((1, C1), lambda i: (0, 0)),
            pl.BlockSpec((1, C1), lambda i: (0, 0)),
            pl.BlockSpec((1, C1), lambda i: (0, 0)),
            pl.BlockSpec((2, C1), lambda i: (0, 0)),
            pl.BlockSpec((C1, C2), lambda i: (0, 0)),
            pl.BlockSpec((1, C2), lambda i: (0, 0)),
        ],
        out_specs=[
            pl.BlockSpec((2, C2), lambda i: (0, 0)),
            pl.BlockSpec((TN // k, C2), lambda i: (i, 0)),
            pl.BlockSpec((TN // k, C2), lambda i: (i, 0)),
        ],
        out_shape=[
            jax.ShapeDtypeStruct((2, C2), jnp.float32),
            jax.ShapeDtypeStruct((N // k, C2), jnp.float32),
            jax.ShapeDtypeStruct((N // k, C2), jnp.float32),
        ],
    )(x, crep, W1pad, W1xT, b1, g1, be1, s1, W2T, b2)


def _final_kernel(ymax_ref, ymin_ref, s2_ref, g2_ref, be2_ref, out_ref, *, N):
    s2 = s2_ref[0:1, :]
    ss2 = s2_ref[1:2, :]
    mean2 = s2 / N
    var2 = ss2 / N - mean2 * mean2
    a2 = g2_ref[...] * lax.rsqrt(var2 + EPS)
    d2 = be2_ref[...] - mean2 * a2
    sel = jnp.where(a2 >= 0.0, ymax_ref[...], ymin_ref[...])
    out_ref[...] = jnp.maximum(sel * a2 + d2, 0.0)


def _run_final(ymax, ymin, s2, g2, be2, N, C2, TF=1024):
    M_all = ymax.shape[0]
    grid = (M_all // TF,)
    return pl.pallas_call(
        functools.partial(_final_kernel, N=N),
        grid=grid,
        in_specs=[
            pl.BlockSpec((TF, C2), lambda i: (i, 0)),
            pl.BlockSpec((TF, C2), lambda i: (i, 0)),
            pl.BlockSpec((2, C2), lambda i: (0, 0)),
            pl.BlockSpec((1, C2), lambda i: (0, 0)),
            pl.BlockSpec((1, C2), lambda i: (0, 0)),
        ],
        out_specs=pl.BlockSpec((TF, C2), lambda i: (i, 0)),
        out_shape=jax.ShapeDtypeStruct((M_all, C2), jnp.float32),
    )(ymax, ymin, s2, g2, be2)


@jax.jit
def kernel(xyz, feats, W1, bi1, g1, be1, W2, bi2, g2, be2):
    B, P, _ = xyz.shape
    C = feats.shape[1]
    M = max(1, P // 4)
    k = min(NSAMPLE, P)
    N = B * M * k
    D = 128
    C1 = W1.shape[0]
    C2 = W2.shape[0]

    idx_center = jnp.linspace(0.0, float(P - 1), M).astype(jnp.int32)
    centers = xyz[:, idx_center, :]

    xyz4 = jnp.transpose(xyz, (0, 2, 1)).reshape(B, 3, NSEG, SEGW)
    idx_g = _run_topk(xyz4, centers, B, M, P, k)
    idx_g = jnp.transpose(idx_g, (0, 2, 1))

    feats_perm = jnp.transpose(feats, (0, 2, 1))
    table = jnp.concatenate(
        [feats_perm, xyz, jnp.zeros((B, P, D - C - 3), jnp.float32)], axis=2
    ).reshape(B * P, D)
    idx_flat = idx_g.reshape(N)
    x = _run_gather(table, idx_flat, N, D)

    W1pad = jnp.concatenate(
        [W1[:, 3:3 + C].T, W1[:, 0:3].T, jnp.zeros((D - C - 3, C1), jnp.float32)],
        axis=0)
    W1xT = W1[:, 0:3].T
    W2T = W2.T
    b1 = bi1.reshape(1, C1)
    g1r = g1.reshape(1, C1)
    be1r = be1.reshape(1, C1)
    b2 = bi2.reshape(1, C2)
    g2r = g2.reshape(1, C2)
    be2r = be2.reshape(1, C2)
    crep = jnp.repeat(centers.reshape(B * M, 3), k, axis=0)

    s1 = _run_stats1(x, crep, W1pad, W1xT, b1, N, D, C1)
    s2, ymax, ymin = _run_layer2(x, crep, W1pad, W1xT, b1, g1r, be1r, s1,
                                 W2T, b2, N, D, C1, C2, k)
    pooled = _run_final(ymax, ymin, s2, g2r, be2r, N, C2)

    out = jnp.transpose(pooled.reshape(B, M, C2), (0, 2, 1))
    return centers, out
